# baseline (speedup 1.0000x reference)
import functools

import jax
import jax.numpy as jnp
from jax import lax
from jax.experimental import pallas as pl
from jax.experimental.pallas import tpu as pltpu
from jax.experimental.pallas import tpu_sc as plsc

N_ROWS = 65536
N_COLS = 1024
ROW_BLOCK = 1024


def _argmax_body(k_ref, idx_ref):
    x = k_ref[...]
    m = jnp.max(x, axis=1, keepdims=True)
    col = lax.broadcasted_iota(jnp.int32, x.shape, 1)
    amax = jnp.min(jnp.where(x == m, col, N_COLS), axis=1)
    row = pl.program_id(0) * ROW_BLOCK + lax.iota(jnp.int32, ROW_BLOCK)
    idx_ref[...] = row * N_COLS + amax


def _tc_argmax(k):
    return pl.pallas_call(
        _argmax_body,
        grid=(N_ROWS // ROW_BLOCK,),
        in_specs=[pl.BlockSpec((ROW_BLOCK, N_COLS), lambda i: (i, 0))],
        out_specs=pl.BlockSpec((ROW_BLOCK,), lambda i: (i,)),
        out_shape=jax.ShapeDtypeStruct((N_ROWS,), jnp.int32),
    )(k)


def _make_sc_gather():
    info = plsc.get_sparse_core_info()
    nw = info.num_cores * info.num_subcores
    b_per_w = N_ROWS // nw
    mesh = plsc.VectorSubcoreMesh(core_axis_name="c", subcore_axis_name="s")

    @functools.partial(
        pl.kernel,
        mesh=mesh,
        out_type=jax.ShapeDtypeStruct((N_ROWS,), jnp.float32),
        scratch_types=[
            pltpu.VMEM((b_per_w,), jnp.int32),
            pltpu.VMEM((b_per_w,), jnp.float32),
            pltpu.SemaphoreType.DMA,
        ],
    )
    def gather(vflat_hbm, idx_hbm, out_hbm, idx_v, vals_v, sem):
        wid = lax.axis_index("s") * info.num_cores + lax.axis_index("c")
        base = wid * b_per_w
        pltpu.sync_copy(idx_hbm.at[pl.ds(base, b_per_w)], idx_v)
        pltpu.async_copy(vflat_hbm.at[idx_v], vals_v, sem).wait()
        pltpu.sync_copy(vals_v, out_hbm.at[pl.ds(base, b_per_w)])

    return gather


_sc_gather = _make_sc_gather()


def kernel(v, k):
    flat_idx = _tc_argmax(k)
    return _sc_gather(v.reshape(-1), flat_idx)


# speedup vs baseline: 2.0614x; 2.0614x over previous
import functools

import jax
import jax.numpy as jnp
from jax import lax
from jax.experimental import pallas as pl
from jax.experimental.pallas import tpu as pltpu
from jax.experimental.pallas import tpu_sc as plsc

N_ROWS = 65536
N_COLS = 1024
ROW_BLOCK = 1024


def _argmax_body(k_ref, idx_ref):
    x = k_ref[...]
    m = jnp.max(x, axis=1, keepdims=True)
    col = lax.broadcasted_iota(jnp.int32, x.shape, 1)
    amax = jnp.min(jnp.where(x == m, col, N_COLS), axis=1)
    row = pl.program_id(0) * ROW_BLOCK + lax.iota(jnp.int32, ROW_BLOCK)
    idx_ref[...] = (
        (row // 8) * 8192
        + (amax // 128) * 1024
        + (row % 8) * 128
        + (amax % 128)
    )


def _tc_argmax(k):
    return pl.pallas_call(
        _argmax_body,
        grid=(N_ROWS // ROW_BLOCK,),
        in_specs=[pl.BlockSpec((ROW_BLOCK, N_COLS), lambda i: (i, 0))],
        out_specs=pl.BlockSpec((ROW_BLOCK,), lambda i: (i,)),
        out_shape=jax.ShapeDtypeStruct((N_ROWS,), jnp.int32),
    )(k)


def _make_sc_gather():
    info = plsc.get_sparse_core_info()
    nw = info.num_cores * info.num_subcores
    b_per_w = N_ROWS // nw
    mesh = plsc.VectorSubcoreMesh(core_axis_name="c", subcore_axis_name="s")

    @functools.partial(
        pl.kernel,
        mesh=mesh,
        out_type=jax.ShapeDtypeStruct((N_ROWS,), jnp.float32),
        scratch_types=[
            pltpu.VMEM((b_per_w,), jnp.int32),
            pltpu.VMEM((b_per_w,), jnp.float32),
            pltpu.SemaphoreType.DMA,
        ],
    )
    def gather(vflat_hbm, idx_hbm, out_hbm, idx_v, vals_v, sem):
        wid = lax.axis_index("s") * info.num_cores + lax.axis_index("c")
        base = wid * b_per_w
        pltpu.sync_copy(idx_hbm.at[pl.ds(base, b_per_w)], idx_v)
        pltpu.async_copy(vflat_hbm.at[idx_v], vals_v, sem).wait()
        pltpu.sync_copy(vals_v, out_hbm.at[pl.ds(base, b_per_w)])

    return gather


_sc_gather = _make_sc_gather()


def kernel(v, k):
    flat_idx = _tc_argmax(k)
    v_view = (
        v.reshape(N_ROWS // 8, 8, N_COLS // 128, 128)
        .transpose(0, 2, 1, 3)
        .reshape(-1)
    )
    return _sc_gather(v_view, flat_idx)


# speedup vs baseline: 2.1762x; 1.0557x over previous
import functools

import jax
import jax.numpy as jnp
from jax import lax
from jax.experimental import pallas as pl
from jax.experimental.pallas import tpu as pltpu
from jax.experimental.pallas import tpu_sc as plsc

N_ROWS = 65536
N_COLS = 1024
ROW_BLOCK = 1024


def _argmax_body(k_ref, idx_ref):
    x = k_ref[...]
    m = jnp.max(x, axis=1, keepdims=True)
    colf = lax.broadcasted_iota(jnp.int32, x.shape, 1).astype(jnp.float32)
    amax = jnp.min(jnp.where(x == m, colf, 2048.0), axis=1).astype(jnp.int32)
    row = pl.program_id(0) * ROW_BLOCK + lax.iota(jnp.int32, ROW_BLOCK)
    idx_ref[...] = (
        (row // 8) * 8192
        + (amax // 128) * 1024
        + (row % 8) * 128
        + (amax % 128)
    )


def _tc_argmax(k):
    return pl.pallas_call(
        _argmax_body,
        grid=(N_ROWS // ROW_BLOCK,),
        in_specs=[pl.BlockSpec((ROW_BLOCK, N_COLS), lambda i: (i, 0))],
        out_specs=pl.BlockSpec((ROW_BLOCK,), lambda i: (i,)),
        out_shape=jax.ShapeDtypeStruct((N_ROWS,), jnp.int32),
    )(k)


def _make_sc_gather():
    info = plsc.get_sparse_core_info()
    nw = info.num_cores * info.num_subcores
    b_per_w = N_ROWS // nw
    mesh = plsc.VectorSubcoreMesh(core_axis_name="c", subcore_axis_name="s")

    @functools.partial(
        pl.kernel,
        mesh=mesh,
        out_type=jax.ShapeDtypeStruct((N_ROWS,), jnp.float32),
        scratch_types=[
            pltpu.VMEM((b_per_w,), jnp.int32),
            pltpu.VMEM((b_per_w,), jnp.float32),
            pltpu.SemaphoreType.DMA,
        ],
    )
    def gather(vflat_hbm, idx_hbm, out_hbm, idx_v, vals_v, sem):
        wid = lax.axis_index("s") * info.num_cores + lax.axis_index("c")
        base = wid * b_per_w
        pltpu.sync_copy(idx_hbm.at[pl.ds(base, b_per_w)], idx_v)
        pltpu.async_copy(vflat_hbm.at[idx_v], vals_v, sem).wait()
        pltpu.sync_copy(vals_v, out_hbm.at[pl.ds(base, b_per_w)])

    return gather


_sc_gather = _make_sc_gather()


def kernel(v, k):
    flat_idx = _tc_argmax(k)
    v_view = (
        v.reshape(N_ROWS // 8, 8, N_COLS // 128, 128)
        .transpose(0, 2, 1, 3)
        .reshape(-1)
    )
    return _sc_gather(v_view, flat_idx)
